# baseline (speedup 1.0000x reference)
import functools
import math

import jax
import jax.numpy as jnp
from jax import lax
from jax.experimental import pallas as pl
from jax.experimental.pallas import tpu as pltpu
from jax.experimental.pallas import tpu_sc as plsc

N = 10000
E = 320000
F_IN = 128
H = 512
C = 10
G = 64

NC = 2
NS = 16
NW = NC * NS
EPW = E // NW
CH = 80
EPW2 = 10240
NCHUNK = EPW2 // CH
BSZ = 32
NBANK = NCHUNK // BSZ
NPAD = 10240
NPT = NPAD // NS
CHP = 64
QV = 1024 // 16

_INV_SQRT = 1.0 / math.sqrt(1.0 + 1e-5)


def _leaky(v):
    return jnp.where(v >= 0, v, 0.01 * v)


def _make_agg(nblk):
    mesh = plsc.VectorSubcoreMesh(core_axis_name="c", subcore_axis_name="s",
                                  num_cores=NC, num_subcores=NS)
    scratch = [
        pltpu.VMEM((BSZ, CH), jnp.int32),
        pltpu.VMEM((BSZ, CH), jnp.int32),
        pltpu.VMEM((CH, F_IN), jnp.float32),
        pltpu.VMEM((CH, F_IN), jnp.float32),
        pltpu.VMEM_SHARED((NPAD, F_IN), jnp.float32),
        pltpu.SemaphoreType.DMA,
        pltpu.SemaphoreType.DMA,
        pltpu.SemaphoreType.DMA,
        pltpu.SemaphoreType.DMA,
    ]
    NP2 = BSZ // 2

    def body(*refs):
        tables = refs[:nblk]
        src_hbm, dst_hbm, zeros_hbm = refs[nblk:nblk + 3]
        out_hbm = refs[nblk + 3]
        src_v, dst_v, rows0, rows1, acc_sh, g0s, g1s, s0s, s1s = refs[nblk + 4:]

        cc = lax.axis_index("c")
        ss = lax.axis_index("s")
        wid = ss * NC + cc
        nbase = ss * NPT

        for b in range(nblk):
            tab = tables[b]

            def gath(c, buf, sem):
                pltpu.async_copy(tab.at[src_v.at[c]], buf, sem)

            def gwait(c, buf, sem):
                pltpu.make_async_copy(tab.at[src_v.at[c]], buf, sem).wait()

            def scat(c, buf, sem):
                pltpu.async_copy(buf, acc_sh.at[dst_v.at[c]], sem, add=True)

            def swait(c, buf, sem):
                pltpu.make_async_copy(buf, acc_sh.at[dst_v.at[c]], sem).wait()

            pltpu.sync_copy(zeros_hbm, acc_sh.at[pl.ds(nbase, NPT)])
            plsc.subcore_barrier()

            for k in range(NBANK):
                pltpu.sync_copy(src_hbm.at[wid, pl.ds(k * BSZ, BSZ)], src_v)
                pltpu.sync_copy(dst_hbm.at[wid, pl.ds(k * BSZ, BSZ)], dst_v)
                gath(0, rows0, g0s)

                def step(i, carry):
                    c0 = 2 * i
                    c1 = c0 + 1
                    gwait(c0, rows0, g0s)
                    scat(c0, rows0, s0s)

                    @pl.when(i > 0)
                    def _():
                        swait(c1 - 2, rows1, s1s)

                    gath(c1, rows1, g1s)
                    gwait(c1, rows1, g1s)
                    scat(c1, rows1, s1s)
                    swait(c0, rows0, s0s)

                    @pl.when(i + 1 < NP2)
                    def _():
                        gath(c0 + 2, rows0, g0s)

                    return carry

                lax.fori_loop(0, NP2, step, 0)
                swait(BSZ - 1, rows1, s1s)

            plsc.subcore_barrier()
            pltpu.sync_copy(acc_sh.at[pl.ds(nbase, NPT)],
                            out_hbm.at[cc, b, pl.ds(nbase, NPT)])
            if b + 1 < nblk:
                plsc.subcore_barrier()

    out_type = jax.ShapeDtypeStruct((NC, nblk, NPAD, F_IN), jnp.float32)
    return pl.kernel(body, out_type=out_type, mesh=mesh, scratch_types=scratch)


_agg1 = _make_agg(1)
_agg4 = _make_agg(4)


def _pool_body(hh_hbm, batch_hbm, out_hbm, batch_v, rows_v, acc_v, sem):
    cc = lax.axis_index("c")
    ss = lax.axis_index("s")
    wid = ss * NC + cc
    g0 = wid * 2

    pltpu.sync_copy(batch_hbm, batch_v.at[pl.ds(0, N)])
    batch_v[pl.ds(N, 16)] = jnp.full((16,), G + 1, jnp.int32)

    def lower_bound(g):
        def bstep(i, lohi):
            lo, hi = lohi
            mid = (lo + hi) // 2
            v = batch_v[pl.ds(mid, 16)][0]
            lo2 = jnp.where(v < g, mid + 1, lo)
            hi2 = jnp.where(v < g, hi, mid)
            return lo2, hi2

        lo, _ = lax.fori_loop(0, 14, bstep, (0, N))
        return lo

    bounds = (lower_bound(g0), lower_bound(g0 + 1), lower_bound(g0 + 2))

    for k in range(2):
        lo = bounds[k]
        hi = bounds[k + 1]
        for q in range(QV):
            acc_v[0, pl.ds(q * 16, 16)] = jnp.full((16,), -jnp.inf, jnp.float32)
        w0 = (lo // 8) * 8
        nch = (hi - w0 + CHP - 1) // CHP

        def chunk_step(j, carry, lo=lo, hi=hi, w0=w0):
            eff = jnp.minimum(w0 + j * CHP, N - CHP)
            pltpu.async_copy(hh_hbm.at[pl.ds(eff, CHP)], rows_v, sem).wait()
            r_lo = jnp.maximum(lo - eff, 0)
            r_hi = jnp.minimum(hi - eff, CHP)

            def row_step(r, c2):
                for q in range(QV):
                    sl = pl.ds(q * 16, 16)
                    acc_v[0, sl] = jnp.maximum(acc_v[0, sl], rows_v[r, sl])
                return c2

            lax.fori_loop(r_lo, r_hi, row_step, 0)
            return carry

        lax.fori_loop(0, nch, chunk_step, 0)
        pltpu.sync_copy(acc_v, out_hbm.at[g0 + k])


_pool = pl.kernel(
    _pool_body,
    out_type=jax.ShapeDtypeStruct((G, 1, 1024), jnp.float32),
    mesh=plsc.VectorSubcoreMesh(core_axis_name="c", subcore_axis_name="s",
                                num_cores=NC, num_subcores=NS),
    scratch_types=[
        pltpu.VMEM((N + 16,), jnp.int32),
        pltpu.VMEM((CHP, 1024), jnp.float32),
        pltpu.VMEM((1, 1024), jnp.float32),
        pltpu.SemaphoreType.DMA,
    ],
)


_R1 = 2000


def _tc1_body(parts_ref, x_ref, wrel_ref, wroot_ref, b_ref, out_ref):
    agg = parts_ref[0] + parts_ref[1]
    y = jnp.dot(agg, wrel_ref[...], preferred_element_type=jnp.float32)
    y = y + jnp.dot(x_ref[...], wroot_ref[...], preferred_element_type=jnp.float32)
    y = _leaky(y + b_ref[...])
    for j in range(4):
        out_ref[j] = y[:, j * 128:(j + 1) * 128]


def _tc1(parts1, x, W1_rel, W1_root, b1):
    return pl.pallas_call(
        _tc1_body,
        grid=(N // _R1,),
        in_specs=[
            pl.BlockSpec((2, _R1, 128), lambda i: (0, i, 0)),
            pl.BlockSpec((_R1, 128), lambda i: (i, 0)),
            pl.BlockSpec((F_IN, H), lambda i: (0, 0)),
            pl.BlockSpec((F_IN, H), lambda i: (0, 0)),
            pl.BlockSpec((1, H), lambda i: (0, 0)),
        ],
        out_specs=pl.BlockSpec((4, _R1, 128), lambda i: (0, i, 0)),
        out_shape=jax.ShapeDtypeStruct((4, N, 128), jnp.float32),
    )(parts1, x, W1_rel, W1_root, b1)


_R2 = 1000


def _tc2_body(x1b_ref, parts_ref, wrel_ref, wroot_ref, b2_ref,
              wl1_ref, bl1_ref, g_ref, be_ref, out_ref):
    acc = None
    for cb in range(4):
        aggc = parts_ref[0, cb] + parts_ref[1, cb]
        t = jnp.dot(aggc, wrel_ref[cb * 128:(cb + 1) * 128, :],
                    preferred_element_type=jnp.float32)
        t = t + jnp.dot(x1b_ref[cb], wroot_ref[cb * 128:(cb + 1) * 128, :],
                        preferred_element_type=jnp.float32)
        acc = t if acc is None else acc + t
    x2 = _leaky(acc + b2_ref[...])
    hacc = jnp.dot(x2, wl1_ref[512:1024, :], preferred_element_type=jnp.float32)
    for cb in range(4):
        hacc = hacc + jnp.dot(x1b_ref[cb], wl1_ref[cb * 128:(cb + 1) * 128, :],
                              preferred_element_type=jnp.float32)
    hv = _leaky(hacc + bl1_ref[...])
    out_ref[...] = g_ref[...] * (hv * _INV_SQRT) + be_ref[...]


def _tc2(x1b, parts2, W2_rel, W2_root, b2, Wl1, bl1, g_l1, be_l1):
    return pl.pallas_call(
        _tc2_body,
        grid=(N // _R2,),
        in_specs=[
            pl.BlockSpec((4, _R2, 128), lambda i: (0, i, 0)),
            pl.BlockSpec((2, 4, _R2, 128), lambda i: (0, 0, i, 0)),
            pl.BlockSpec((H, H), lambda i: (0, 0)),
            pl.BlockSpec((H, H), lambda i: (0, 0)),
            pl.BlockSpec((1, H), lambda i: (0, 0)),
            pl.BlockSpec((2 * H, 1024), lambda i: (0, 0)),
            pl.BlockSpec((1, 1024), lambda i: (0, 0)),
            pl.BlockSpec((1, 1024), lambda i: (0, 0)),
            pl.BlockSpec((1, 1024), lambda i: (0, 0)),
        ],
        out_specs=pl.BlockSpec((_R2, 1024), lambda i: (i, 0)),
        out_shape=jax.ShapeDtypeStruct((N, 1024), jnp.float32),
    )(x1b, parts2, W2_rel, W2_root, b2, Wl1, bl1, g_l1, be_l1)


def _tc3_body(p_ref, wa_ref, ba_ref, ga_ref, bea_ref,
              wb_ref, bb_ref, gb_ref, beb_ref,
              wc_ref, bc_ref, gc_ref, bec_ref, out_ref):
    def bn(v, g, b):
        return g * (v * _INV_SQRT) + b

    o = bn(_leaky(jnp.dot(p_ref[...], wa_ref[...],
                          preferred_element_type=jnp.float32) + ba_ref[...]),
           ga_ref[...], bea_ref[...])
    o = bn(_leaky(jnp.dot(o, wb_ref[...],
                          preferred_element_type=jnp.float32) + bb_ref[...]),
           gb_ref[...], beb_ref[...])
    o = bn(_leaky(jnp.dot(o, wc_ref[...],
                          preferred_element_type=jnp.float32) + bc_ref[...]),
           gc_ref[...], bec_ref[...])
    out_ref[...] = o


def _tc3(pooled, Wa, ba, ga, bea, Wb, bb, gb, beb, Wcp, bcp, gcp, becp):
    return pl.pallas_call(
        _tc3_body,
        out_shape=jax.ShapeDtypeStruct((G, 128), jnp.float32),
    )(pooled, Wa, ba, ga, bea, Wb, bb, gb, beb, Wcp, bcp, gcp, becp)


def kernel(x, edge_index, batch, W1_rel, W1_root, b1, W2_rel, W2_root, b2,
           Wl1, bl1, g_l1, be_l1, Wa, ba, ga, bea, Wb, bb, gb, beb,
           Wc, bc, gc, bec):
    pad = jnp.zeros((NW, EPW2 - EPW), jnp.int32)
    src2 = jnp.concatenate(
        [edge_index[0].reshape(NW, EPW), pad], axis=1).reshape(NW, NCHUNK, CH)
    dst2 = jnp.concatenate(
        [edge_index[1].reshape(NW, EPW), pad + (NPAD - 1)],
        axis=1).reshape(NW, NCHUNK, CH)
    zeros = jnp.zeros((NPT, F_IN), jnp.float32)

    parts1 = _agg1(x, src2, dst2, zeros)
    x1b = _tc1(parts1.reshape(NC, NPAD, F_IN), x,
               W1_rel, W1_root, b1.reshape(1, H))
    parts2 = _agg4(x1b[0], x1b[1], x1b[2], x1b[3],
                   src2, dst2, zeros)
    hh = _tc2(x1b, parts2, W2_rel, W2_root, b2.reshape(1, H),
              Wl1, bl1.reshape(1, 1024), g_l1.reshape(1, 1024),
              be_l1.reshape(1, 1024))
    pooled = _pool(hh, batch).reshape(G, 1024)

    Wcp = jnp.pad(Wc, ((0, 0), (0, 128 - C)))
    bcp = jnp.pad(bc, (0, 128 - C)).reshape(1, 128)
    gcp = jnp.pad(gc, (0, 128 - C), constant_values=1.0).reshape(1, 128)
    becp = jnp.pad(bec, (0, 128 - C)).reshape(1, 128)
    o = _tc3(pooled, Wa, ba.reshape(1, 512), ga.reshape(1, 512),
             bea.reshape(1, 512), Wb, bb.reshape(1, 256), gb.reshape(1, 256),
             beb.reshape(1, 256), Wcp, bcp, gcp, becp)[:, :C]
    return (o, pooled)


# speedup vs baseline: 1.0595x; 1.0595x over previous
import functools
import math

import jax
import jax.numpy as jnp
from jax import lax
from jax.experimental import pallas as pl
from jax.experimental.pallas import tpu as pltpu
from jax.experimental.pallas import tpu_sc as plsc

N = 10000
E = 320000
F_IN = 128
H = 512
C = 10
G = 64

NC = 2
NS = 16
NW = NC * NS
EPW = E // NW
CH = 128
EPW2 = 10240
NCHUNK = EPW2 // CH
BSZ = 40
NBANK = NCHUNK // BSZ
NPAD = 10240
NPT = NPAD // NS
CHP = 64
QV = 1024 // 16

_INV_SQRT = 1.0 / math.sqrt(1.0 + 1e-5)


def _leaky(v):
    return jnp.where(v >= 0, v, 0.01 * v)


def _make_agg(nblk):
    mesh = plsc.VectorSubcoreMesh(core_axis_name="c", subcore_axis_name="s",
                                  num_cores=NC, num_subcores=NS)
    scratch = [
        pltpu.VMEM((BSZ, CH), jnp.int32),
        pltpu.VMEM((BSZ, CH), jnp.int32),
        pltpu.VMEM((CH, F_IN), jnp.float32),
        pltpu.VMEM((CH, F_IN), jnp.float32),
        pltpu.VMEM_SHARED((NPAD, F_IN), jnp.float32),
        pltpu.SemaphoreType.DMA,
        pltpu.SemaphoreType.DMA,
        pltpu.SemaphoreType.DMA,
        pltpu.SemaphoreType.DMA,
    ]
    NP2 = BSZ // 2

    def body(*refs):
        tables = refs[:nblk]
        src_hbm, dst_hbm, zeros_hbm = refs[nblk:nblk + 3]
        out_hbm = refs[nblk + 3]
        src_v, dst_v, rows0, rows1, acc_sh, g0s, g1s, s0s, s1s = refs[nblk + 4:]

        cc = lax.axis_index("c")
        ss = lax.axis_index("s")
        wid = ss * NC + cc
        nbase = ss * NPT

        for b in range(nblk):
            tab = tables[b]

            def gath(c, buf, sem):
                pltpu.async_copy(tab.at[src_v.at[c]], buf, sem)

            def gwait(c, buf, sem):
                pltpu.make_async_copy(tab.at[pl.ds(0, CH)], buf, sem).wait()

            def scat(c, buf, sem):
                pltpu.async_copy(buf, acc_sh.at[dst_v.at[c]], sem, add=True)

            def swait(c, buf, sem):
                pltpu.make_async_copy(buf, acc_sh.at[pl.ds(0, CH)], sem).wait()

            pltpu.sync_copy(zeros_hbm, acc_sh.at[pl.ds(nbase, NPT)])
            plsc.subcore_barrier()

            for k in range(NBANK):
                pltpu.sync_copy(src_hbm.at[wid, pl.ds(k * BSZ, BSZ)], src_v)
                pltpu.sync_copy(dst_hbm.at[wid, pl.ds(k * BSZ, BSZ)], dst_v)
                gath(0, rows0, g0s)

                def step(i, carry):
                    c0 = 2 * i
                    c1 = c0 + 1
                    gwait(c0, rows0, g0s)
                    scat(c0, rows0, s0s)

                    @pl.when(i > 0)
                    def _():
                        swait(c1 - 2, rows1, s1s)

                    gath(c1, rows1, g1s)
                    gwait(c1, rows1, g1s)
                    scat(c1, rows1, s1s)
                    swait(c0, rows0, s0s)

                    @pl.when(i + 1 < NP2)
                    def _():
                        gath(c0 + 2, rows0, g0s)

                    return carry

                lax.fori_loop(0, NP2, step, 0)
                swait(BSZ - 1, rows1, s1s)

            plsc.subcore_barrier()
            pltpu.sync_copy(acc_sh.at[pl.ds(nbase, NPT)],
                            out_hbm.at[cc, b, pl.ds(nbase, NPT)])
            if b + 1 < nblk:
                plsc.subcore_barrier()

    out_type = jax.ShapeDtypeStruct((NC, nblk, NPAD, F_IN), jnp.float32)
    return pl.kernel(body, out_type=out_type, mesh=mesh, scratch_types=scratch)


_agg1 = _make_agg(1)
_agg4 = _make_agg(4)


def _pool_body(hh_hbm, batch_hbm, out_hbm, batch_v, rows_v, acc_v, sem):
    cc = lax.axis_index("c")
    ss = lax.axis_index("s")
    wid = ss * NC + cc
    g0 = wid * 2

    pltpu.sync_copy(batch_hbm, batch_v.at[pl.ds(0, N)])
    batch_v[pl.ds(N, 16)] = jnp.full((16,), G + 1, jnp.int32)

    def lower_bound(g):
        def bstep(i, lohi):
            lo, hi = lohi
            mid = (lo + hi) // 2
            v = batch_v[pl.ds(mid, 16)][0]
            lo2 = jnp.where(v < g, mid + 1, lo)
            hi2 = jnp.where(v < g, hi, mid)
            return lo2, hi2

        lo, _ = lax.fori_loop(0, 14, bstep, (0, N))
        return lo

    bounds = (lower_bound(g0), lower_bound(g0 + 1), lower_bound(g0 + 2))

    for k in range(2):
        lo = bounds[k]
        hi = bounds[k + 1]
        for q in range(QV):
            acc_v[0, pl.ds(q * 16, 16)] = jnp.full((16,), -jnp.inf, jnp.float32)
        w0 = (lo // 8) * 8
        nch = (hi - w0 + CHP - 1) // CHP

        def chunk_step(j, carry, lo=lo, hi=hi, w0=w0):
            eff = jnp.minimum(w0 + j * CHP, N - CHP)
            pltpu.async_copy(hh_hbm.at[pl.ds(eff, CHP)], rows_v, sem).wait()
            r_lo = jnp.maximum(lo - eff, 0)
            r_hi = jnp.minimum(hi - eff, CHP)

            def row_step(r, c2):
                for q in range(QV):
                    sl = pl.ds(q * 16, 16)
                    acc_v[0, sl] = jnp.maximum(acc_v[0, sl], rows_v[r, sl])
                return c2

            lax.fori_loop(r_lo, r_hi, row_step, 0)
            return carry

        lax.fori_loop(0, nch, chunk_step, 0)
        pltpu.sync_copy(acc_v, out_hbm.at[g0 + k])


_pool = pl.kernel(
    _pool_body,
    out_type=jax.ShapeDtypeStruct((G, 1, 1024), jnp.float32),
    mesh=plsc.VectorSubcoreMesh(core_axis_name="c", subcore_axis_name="s",
                                num_cores=NC, num_subcores=NS),
    scratch_types=[
        pltpu.VMEM((N + 16,), jnp.int32),
        pltpu.VMEM((CHP, 1024), jnp.float32),
        pltpu.VMEM((1, 1024), jnp.float32),
        pltpu.SemaphoreType.DMA,
    ],
)


_R1 = 2000


def _tc1_body(parts_ref, x_ref, wrel_ref, wroot_ref, b_ref, out_ref):
    agg = parts_ref[0] + parts_ref[1]
    y = jnp.dot(agg, wrel_ref[...], preferred_element_type=jnp.float32)
    y = y + jnp.dot(x_ref[...], wroot_ref[...], preferred_element_type=jnp.float32)
    y = _leaky(y + b_ref[...])
    for j in range(4):
        out_ref[j] = y[:, j * 128:(j + 1) * 128]


def _tc1(parts1, x, W1_rel, W1_root, b1):
    return pl.pallas_call(
        _tc1_body,
        grid=(N // _R1,),
        in_specs=[
            pl.BlockSpec((2, _R1, 128), lambda i: (0, i, 0)),
            pl.BlockSpec((_R1, 128), lambda i: (i, 0)),
            pl.BlockSpec((F_IN, H), lambda i: (0, 0)),
            pl.BlockSpec((F_IN, H), lambda i: (0, 0)),
            pl.BlockSpec((1, H), lambda i: (0, 0)),
        ],
        out_specs=pl.BlockSpec((4, _R1, 128), lambda i: (0, i, 0)),
        out_shape=jax.ShapeDtypeStruct((4, N, 128), jnp.float32),
    )(parts1, x, W1_rel, W1_root, b1)


_R2 = 1000


def _tc2_body(x1b_ref, parts_ref, wrel_ref, wroot_ref, b2_ref,
              wl1_ref, bl1_ref, g_ref, be_ref, out_ref):
    acc = None
    for cb in range(4):
        aggc = parts_ref[0, cb] + parts_ref[1, cb]
        t = jnp.dot(aggc, wrel_ref[cb * 128:(cb + 1) * 128, :],
                    preferred_element_type=jnp.float32)
        t = t + jnp.dot(x1b_ref[cb], wroot_ref[cb * 128:(cb + 1) * 128, :],
                        preferred_element_type=jnp.float32)
        acc = t if acc is None else acc + t
    x2 = _leaky(acc + b2_ref[...])
    hacc = jnp.dot(x2, wl1_ref[512:1024, :], preferred_element_type=jnp.float32)
    for cb in range(4):
        hacc = hacc + jnp.dot(x1b_ref[cb], wl1_ref[cb * 128:(cb + 1) * 128, :],
                              preferred_element_type=jnp.float32)
    hv = _leaky(hacc + bl1_ref[...])
    out_ref[...] = g_ref[...] * (hv * _INV_SQRT) + be_ref[...]


def _tc2(x1b, parts2, W2_rel, W2_root, b2, Wl1, bl1, g_l1, be_l1):
    return pl.pallas_call(
        _tc2_body,
        grid=(N // _R2,),
        in_specs=[
            pl.BlockSpec((4, _R2, 128), lambda i: (0, i, 0)),
            pl.BlockSpec((2, 4, _R2, 128), lambda i: (0, 0, i, 0)),
            pl.BlockSpec((H, H), lambda i: (0, 0)),
            pl.BlockSpec((H, H), lambda i: (0, 0)),
            pl.BlockSpec((1, H), lambda i: (0, 0)),
            pl.BlockSpec((2 * H, 1024), lambda i: (0, 0)),
            pl.BlockSpec((1, 1024), lambda i: (0, 0)),
            pl.BlockSpec((1, 1024), lambda i: (0, 0)),
            pl.BlockSpec((1, 1024), lambda i: (0, 0)),
        ],
        out_specs=pl.BlockSpec((_R2, 1024), lambda i: (i, 0)),
        out_shape=jax.ShapeDtypeStruct((N, 1024), jnp.float32),
    )(x1b, parts2, W2_rel, W2_root, b2, Wl1, bl1, g_l1, be_l1)


def _tc3_body(p_ref, wa_ref, ba_ref, ga_ref, bea_ref,
              wb_ref, bb_ref, gb_ref, beb_ref,
              wc_ref, bc_ref, gc_ref, bec_ref, out_ref):
    def bn(v, g, b):
        return g * (v * _INV_SQRT) + b

    o = bn(_leaky(jnp.dot(p_ref[...], wa_ref[...],
                          preferred_element_type=jnp.float32) + ba_ref[...]),
           ga_ref[...], bea_ref[...])
    o = bn(_leaky(jnp.dot(o, wb_ref[...],
                          preferred_element_type=jnp.float32) + bb_ref[...]),
           gb_ref[...], beb_ref[...])
    o = bn(_leaky(jnp.dot(o, wc_ref[...],
                          preferred_element_type=jnp.float32) + bc_ref[...]),
           gc_ref[...], bec_ref[...])
    out_ref[...] = o


def _tc3(pooled, Wa, ba, ga, bea, Wb, bb, gb, beb, Wcp, bcp, gcp, becp):
    return pl.pallas_call(
        _tc3_body,
        out_shape=jax.ShapeDtypeStruct((G, 128), jnp.float32),
    )(pooled, Wa, ba, ga, bea, Wb, bb, gb, beb, Wcp, bcp, gcp, becp)


def kernel(x, edge_index, batch, W1_rel, W1_root, b1, W2_rel, W2_root, b2,
           Wl1, bl1, g_l1, be_l1, Wa, ba, ga, bea, Wb, bb, gb, beb,
           Wc, bc, gc, bec):
    pad = jnp.zeros((NW, EPW2 - EPW), jnp.int32)
    src2 = jnp.concatenate(
        [edge_index[0].reshape(NW, EPW), pad], axis=1).reshape(NW, NCHUNK, CH)
    dst2 = jnp.concatenate(
        [edge_index[1].reshape(NW, EPW), pad + (NPAD - 1)],
        axis=1).reshape(NW, NCHUNK, CH)
    zeros = jnp.zeros((NPT, F_IN), jnp.float32)

    parts1 = _agg1(x, src2, dst2, zeros)
    x1b = _tc1(parts1.reshape(NC, NPAD, F_IN), x,
               W1_rel, W1_root, b1.reshape(1, H))
    parts2 = _agg4(x1b[0], x1b[1], x1b[2], x1b[3],
                   src2, dst2, zeros)
    hh = _tc2(x1b, parts2, W2_rel, W2_root, b2.reshape(1, H),
              Wl1, bl1.reshape(1, 1024), g_l1.reshape(1, 1024),
              be_l1.reshape(1, 1024))
    pooled = _pool(hh, batch).reshape(G, 1024)

    Wcp = jnp.pad(Wc, ((0, 0), (0, 128 - C)))
    bcp = jnp.pad(bc, (0, 128 - C)).reshape(1, 128)
    gcp = jnp.pad(gc, (0, 128 - C), constant_values=1.0).reshape(1, 128)
    becp = jnp.pad(bec, (0, 128 - C)).reshape(1, 128)
    o = _tc3(pooled, Wa, ba.reshape(1, 512), ga.reshape(1, 512),
             bea.reshape(1, 512), Wb, bb.reshape(1, 256), gb.reshape(1, 256),
             beb.reshape(1, 256), Wcp, bcp, gcp, becp)[:, :C]
    return (o, pooled)


# speedup vs baseline: 1.4895x; 1.4058x over previous
import functools
import math

import jax
import jax.numpy as jnp
from jax import lax
from jax.experimental import pallas as pl
from jax.experimental.pallas import tpu as pltpu
from jax.experimental.pallas import tpu_sc as plsc

N = 10000
E = 320000
F_IN = 128
H = 512
C = 10
G = 64

NC = 2
NS = 16
NW = NC * NS
EPW = E // NW
CH = 128
EPW2 = 10240
NCHUNK = EPW2 // CH
BSZ = 40
NBANK = NCHUNK // BSZ
NPAD = 10240
NPT = NPAD // NS
CHP = 64
QV = 1024 // 16

_INV_SQRT = 1.0 / math.sqrt(1.0 + 1e-5)


def _leaky(v):
    return jnp.where(v >= 0, v, 0.01 * v)


_SC_MESH = plsc.VectorSubcoreMesh(core_axis_name="c", subcore_axis_name="s",
                                  num_cores=NC, num_subcores=NS)
_AGG_SCRATCH = [
    pltpu.VMEM((BSZ, CH), jnp.int32),
    pltpu.VMEM((BSZ, CH), jnp.int32),
    pltpu.VMEM((CH, F_IN), jnp.float32),
    pltpu.VMEM((CH, F_IN), jnp.float32),
    pltpu.VMEM_SHARED((NPAD, F_IN), jnp.float32),
    pltpu.SemaphoreType.DMA,
    pltpu.SemaphoreType.DMA,
]
_GRP = 10
_NGRP = BSZ // _GRP


def _edge_pass(tab, w, src_hbm, dst_hbm, acc_sh,
               src_v, dst_v, rows0, rows1, g0s, g1s):
    rows = (rows0, rows1)
    sems = (g0s, g1s)

    for k in range(NBANK):
        pltpu.sync_copy(src_hbm.at[w, pl.ds(k * BSZ, BSZ)], src_v)
        pltpu.sync_copy(dst_hbm.at[w, pl.ds(k * BSZ, BSZ)], dst_v)

        def group(g, carry):
            base = g * _GRP
            d = [None, None]
            d[0] = pltpu.async_copy(tab.at[src_v.at[base]], rows[0], sems[0])
            for j in range(_GRP):
                if j + 1 < _GRP:
                    d[(j + 1) % 2] = pltpu.async_copy(
                        tab.at[src_v.at[base + j + 1]],
                        rows[(j + 1) % 2], sems[(j + 1) % 2])
                d[j % 2].wait()
                pltpu.sync_copy(rows[j % 2],
                                acc_sh.at[dst_v.at[base + j]], add=True)
            return carry

        lax.fori_loop(0, _NGRP, group, 0)


def _agg1_body(tab, src_hbm, dst_hbm, zeros_hbm, out_hbm,
               src_v, dst_v, rows0, rows1, acc_sh, g0s, g1s):
    cc = lax.axis_index("c")
    ss = lax.axis_index("s")
    wid = ss * NC + cc
    nbase = ss * NPT
    pltpu.sync_copy(zeros_hbm, acc_sh.at[pl.ds(nbase, NPT)])
    plsc.subcore_barrier()
    _edge_pass(tab, wid, src_hbm, dst_hbm, acc_sh,
               src_v, dst_v, rows0, rows1, g0s, g1s)
    plsc.subcore_barrier()
    pltpu.sync_copy(acc_sh.at[pl.ds(nbase, NPT)],
                    out_hbm.at[cc, pl.ds(nbase, NPT)])


_agg1 = pl.kernel(
    _agg1_body,
    out_type=jax.ShapeDtypeStruct((NC, NPAD, F_IN), jnp.float32),
    mesh=_SC_MESH, scratch_types=_AGG_SCRATCH)


def _agg4_body(t0, t1, t2, t3, src_hbm, dst_hbm, zeros_hbm, out_hbm,
               src_v, dst_v, rows0, rows1, acc_sh, g0s, g1s):
    cc = lax.axis_index("c")
    ss = lax.axis_index("s")
    nbase = ss * NPT
    tabs = (t0, t1, t2, t3)

    def do_block(b):
        tab = tabs[b]
        pltpu.sync_copy(zeros_hbm, acc_sh.at[pl.ds(nbase, NPT)])
        plsc.subcore_barrier()
        for half in range(2):
            _edge_pass(tab, ss * 2 + half, src_hbm, dst_hbm, acc_sh,
                       src_v, dst_v, rows0, rows1, g0s, g1s)
        plsc.subcore_barrier()
        pltpu.sync_copy(acc_sh.at[pl.ds(nbase, NPT)],
                        out_hbm.at[b, pl.ds(nbase, NPT)])
        plsc.subcore_barrier()

    @pl.when(cc == 0)
    def _():
        do_block(0)
        do_block(1)

    @pl.when(cc == 1)
    def _():
        do_block(2)
        do_block(3)


_agg4 = pl.kernel(
    _agg4_body,
    out_type=jax.ShapeDtypeStruct((4, NPAD, F_IN), jnp.float32),
    mesh=_SC_MESH, scratch_types=_AGG_SCRATCH)


def _pool_body(hh_hbm, batch_hbm, out_hbm, batch_v, rows_v, acc_v, sem):
    cc = lax.axis_index("c")
    ss = lax.axis_index("s")
    wid = ss * NC + cc
    g0 = wid * 2

    pltpu.sync_copy(batch_hbm, batch_v.at[pl.ds(0, N)])
    batch_v[pl.ds(N, 16)] = jnp.full((16,), G + 1, jnp.int32)

    def lower_bound(g):
        def bstep(i, lohi):
            lo, hi = lohi
            mid = (lo + hi) // 2
            v = batch_v[pl.ds(mid, 16)][0]
            lo2 = jnp.where(v < g, mid + 1, lo)
            hi2 = jnp.where(v < g, hi, mid)
            return lo2, hi2

        lo, _ = lax.fori_loop(0, 14, bstep, (0, N))
        return lo

    bounds = (lower_bound(g0), lower_bound(g0 + 1), lower_bound(g0 + 2))

    for k in range(2):
        lo = bounds[k]
        hi = bounds[k + 1]
        for q in range(QV):
            acc_v[0, pl.ds(q * 16, 16)] = jnp.full((16,), -jnp.inf, jnp.float32)
        w0 = (lo // 8) * 8
        nch = (hi - w0 + CHP - 1) // CHP

        def chunk_step(j, carry, lo=lo, hi=hi, w0=w0):
            eff = jnp.minimum(w0 + j * CHP, N - CHP)
            pltpu.async_copy(hh_hbm.at[pl.ds(eff, CHP)], rows_v, sem).wait()
            r_lo = jnp.maximum(lo - eff, 0)
            r_hi = jnp.minimum(hi - eff, CHP)

            def row_step(r, c2):
                for q in range(QV):
                    sl = pl.ds(q * 16, 16)
                    acc_v[0, sl] = jnp.maximum(acc_v[0, sl], rows_v[r, sl])
                return c2

            lax.fori_loop(r_lo, r_hi, row_step, 0)
            return carry

        lax.fori_loop(0, nch, chunk_step, 0)
        pltpu.sync_copy(acc_v, out_hbm.at[g0 + k])


_pool = pl.kernel(
    _pool_body,
    out_type=jax.ShapeDtypeStruct((G, 1, 1024), jnp.float32),
    mesh=plsc.VectorSubcoreMesh(core_axis_name="c", subcore_axis_name="s",
                                num_cores=NC, num_subcores=NS),
    scratch_types=[
        pltpu.VMEM((N + 16,), jnp.int32),
        pltpu.VMEM((CHP, 1024), jnp.float32),
        pltpu.VMEM((1, 1024), jnp.float32),
        pltpu.SemaphoreType.DMA,
    ],
)


_R1 = 2000


def _tc1_body(parts_ref, x_ref, wrel_ref, wroot_ref, b_ref, out_ref):
    agg = parts_ref[0] + parts_ref[1]
    y = jnp.dot(agg, wrel_ref[...], preferred_element_type=jnp.float32)
    y = y + jnp.dot(x_ref[...], wroot_ref[...], preferred_element_type=jnp.float32)
    y = _leaky(y + b_ref[...])
    for j in range(4):
        out_ref[j] = y[:, j * 128:(j + 1) * 128]


def _tc1(parts1, x, W1_rel, W1_root, b1):
    return pl.pallas_call(
        _tc1_body,
        grid=(N // _R1,),
        in_specs=[
            pl.BlockSpec((2, _R1, 128), lambda i: (0, i, 0)),
            pl.BlockSpec((_R1, 128), lambda i: (i, 0)),
            pl.BlockSpec((F_IN, H), lambda i: (0, 0)),
            pl.BlockSpec((F_IN, H), lambda i: (0, 0)),
            pl.BlockSpec((1, H), lambda i: (0, 0)),
        ],
        out_specs=pl.BlockSpec((4, _R1, 128), lambda i: (0, i, 0)),
        out_shape=jax.ShapeDtypeStruct((4, N, 128), jnp.float32),
    )(parts1, x, W1_rel, W1_root, b1)


_R2 = 1000


def _tc2_body(x1b_ref, parts_ref, wrel_ref, wroot_ref, b2_ref,
              wl1_ref, bl1_ref, g_ref, be_ref, out_ref):
    acc = None
    for cb in range(4):
        aggc = parts_ref[cb]
        t = jnp.dot(aggc, wrel_ref[cb * 128:(cb + 1) * 128, :],
                    preferred_element_type=jnp.float32)
        t = t + jnp.dot(x1b_ref[cb], wroot_ref[cb * 128:(cb + 1) * 128, :],
                        preferred_element_type=jnp.float32)
        acc = t if acc is None else acc + t
    x2 = _leaky(acc + b2_ref[...])
    hacc = jnp.dot(x2, wl1_ref[512:1024, :], preferred_element_type=jnp.float32)
    for cb in range(4):
        hacc = hacc + jnp.dot(x1b_ref[cb], wl1_ref[cb * 128:(cb + 1) * 128, :],
                              preferred_element_type=jnp.float32)
    hv = _leaky(hacc + bl1_ref[...])
    out_ref[...] = g_ref[...] * (hv * _INV_SQRT) + be_ref[...]


def _tc2(x1b, parts2, W2_rel, W2_root, b2, Wl1, bl1, g_l1, be_l1):
    return pl.pallas_call(
        _tc2_body,
        grid=(N // _R2,),
        in_specs=[
            pl.BlockSpec((4, _R2, 128), lambda i: (0, i, 0)),
            pl.BlockSpec((4, _R2, 128), lambda i: (0, i, 0)),
            pl.BlockSpec((H, H), lambda i: (0, 0)),
            pl.BlockSpec((H, H), lambda i: (0, 0)),
            pl.BlockSpec((1, H), lambda i: (0, 0)),
            pl.BlockSpec((2 * H, 1024), lambda i: (0, 0)),
            pl.BlockSpec((1, 1024), lambda i: (0, 0)),
            pl.BlockSpec((1, 1024), lambda i: (0, 0)),
            pl.BlockSpec((1, 1024), lambda i: (0, 0)),
        ],
        out_specs=pl.BlockSpec((_R2, 1024), lambda i: (i, 0)),
        out_shape=jax.ShapeDtypeStruct((N, 1024), jnp.float32),
    )(x1b, parts2, W2_rel, W2_root, b2, Wl1, bl1, g_l1, be_l1)


def _tc3_body(p_ref, wa_ref, ba_ref, ga_ref, bea_ref,
              wb_ref, bb_ref, gb_ref, beb_ref,
              wc_ref, bc_ref, gc_ref, bec_ref, out_ref):
    def bn(v, g, b):
        return g * (v * _INV_SQRT) + b

    o = bn(_leaky(jnp.dot(p_ref[...], wa_ref[...],
                          preferred_element_type=jnp.float32) + ba_ref[...]),
           ga_ref[...], bea_ref[...])
    o = bn(_leaky(jnp.dot(o, wb_ref[...],
                          preferred_element_type=jnp.float32) + bb_ref[...]),
           gb_ref[...], beb_ref[...])
    o = bn(_leaky(jnp.dot(o, wc_ref[...],
                          preferred_element_type=jnp.float32) + bc_ref[...]),
           gc_ref[...], bec_ref[...])
    out_ref[...] = o


def _tc3(pooled, Wa, ba, ga, bea, Wb, bb, gb, beb, Wcp, bcp, gcp, becp):
    return pl.pallas_call(
        _tc3_body,
        out_shape=jax.ShapeDtypeStruct((G, 128), jnp.float32),
    )(pooled, Wa, ba, ga, bea, Wb, bb, gb, beb, Wcp, bcp, gcp, becp)


def kernel(x, edge_index, batch, W1_rel, W1_root, b1, W2_rel, W2_root, b2,
           Wl1, bl1, g_l1, be_l1, Wa, ba, ga, bea, Wb, bb, gb, beb,
           Wc, bc, gc, bec):
    pad = jnp.zeros((NW, EPW2 - EPW), jnp.int32)
    src2 = jnp.concatenate(
        [edge_index[0].reshape(NW, EPW), pad], axis=1).reshape(NW, NCHUNK, CH)
    dst2 = jnp.concatenate(
        [edge_index[1].reshape(NW, EPW), pad + (NPAD - 1)],
        axis=1).reshape(NW, NCHUNK, CH)
    zeros = jnp.zeros((NPT, F_IN), jnp.float32)

    parts1 = _agg1(x, src2, dst2, zeros)
    x1b = _tc1(parts1, x,
               W1_rel, W1_root, b1.reshape(1, H))
    parts2 = _agg4(x1b[0], x1b[1], x1b[2], x1b[3],
                   src2, dst2, zeros)
    hh = _tc2(x1b, parts2, W2_rel, W2_root, b2.reshape(1, H),
              Wl1, bl1.reshape(1, 1024), g_l1.reshape(1, 1024),
              be_l1.reshape(1, 1024))
    pooled = _pool(hh, batch).reshape(G, 1024)

    Wcp = jnp.pad(Wc, ((0, 0), (0, 128 - C)))
    bcp = jnp.pad(bc, (0, 128 - C)).reshape(1, 128)
    gcp = jnp.pad(gc, (0, 128 - C), constant_values=1.0).reshape(1, 128)
    becp = jnp.pad(bec, (0, 128 - C)).reshape(1, 128)
    o = _tc3(pooled, Wa, ba.reshape(1, 512), ga.reshape(1, 512),
             bea.reshape(1, 512), Wb, bb.reshape(1, 256), gb.reshape(1, 256),
             beb.reshape(1, 256), Wcp, bcp, gcp, becp)[:, :C]
    return (o, pooled)


# speedup vs baseline: 1.9783x; 1.3282x over previous
import functools
import math

import jax
import jax.numpy as jnp
from jax import lax
from jax.experimental import pallas as pl
from jax.experimental.pallas import tpu as pltpu
from jax.experimental.pallas import tpu_sc as plsc

N = 10000
E = 320000
F_IN = 128
H = 512
C = 10
G = 64

NC = 2
NS = 16
NW = NC * NS
EPW = E // NW
CH = 80
NCHUNK = EPW // CH
NPAD = 10240
NPT = NPAD // NS
CHP = 48
QV = 1024 // 16

_INV_SQRT = 1.0 / math.sqrt(1.0 + 1e-5)


def _leaky(v):
    return jnp.where(v >= 0, v, 0.01 * v)


_SC_MESH = plsc.VectorSubcoreMesh(core_axis_name="c", subcore_axis_name="s",
                                  num_cores=NC, num_subcores=NS)
_AGG_SCRATCH = [
    pltpu.VMEM((NCHUNK, CH), jnp.int32),
    pltpu.VMEM((NCHUNK, CH), jnp.int32),
    pltpu.VMEM((CH, F_IN), jnp.float32),
    pltpu.VMEM_SHARED((NPAD, F_IN), jnp.float32),
    pltpu.SemaphoreType.DMA,
]


def _edge_pass(tab, w, src_hbm, dst_hbm, acc_sh, src_v, dst_v, rows_v, sem):
    pltpu.sync_copy(src_hbm.at[w], src_v)
    pltpu.sync_copy(dst_hbm.at[w], dst_v)

    def step(i, carry):
        pltpu.async_copy(tab.at[src_v.at[i]], rows_v, sem).wait()
        pltpu.sync_copy(rows_v, acc_sh.at[dst_v.at[i]], add=True)
        return carry

    lax.fori_loop(0, NCHUNK, step, 0)


def _agg1_body(tab, src_hbm, dst_hbm, zeros_hbm, out_hbm,
               src_v, dst_v, rows_v, acc_sh, sem):
    cc = lax.axis_index("c")
    ss = lax.axis_index("s")
    wid = ss * NC + cc
    nbase = ss * NPT
    pltpu.sync_copy(zeros_hbm, acc_sh.at[pl.ds(nbase, NPT)])
    plsc.subcore_barrier()
    _edge_pass(tab, wid, src_hbm, dst_hbm, acc_sh, src_v, dst_v, rows_v, sem)
    plsc.subcore_barrier()
    pltpu.sync_copy(acc_sh.at[pl.ds(nbase, NPT)],
                    out_hbm.at[cc, pl.ds(nbase, NPT)])


_agg1 = pl.kernel(
    _agg1_body,
    out_type=jax.ShapeDtypeStruct((NC, NPAD, F_IN), jnp.float32),
    mesh=_SC_MESH, scratch_types=_AGG_SCRATCH)


def _agg4_body(t0, t1, t2, t3, src_hbm, dst_hbm, zeros_hbm, out_hbm,
               src_v, dst_v, rows_v, acc_sh, sem):
    cc = lax.axis_index("c")
    ss = lax.axis_index("s")
    nbase = ss * NPT
    tabs = (t0, t1, t2, t3)

    def do_block(b):
        tab = tabs[b]
        pltpu.sync_copy(zeros_hbm, acc_sh.at[pl.ds(nbase, NPT)])
        plsc.subcore_barrier()
        for half in range(2):
            _edge_pass(tab, ss * 2 + half, src_hbm, dst_hbm, acc_sh,
                       src_v, dst_v, rows_v, sem)
        plsc.subcore_barrier()
        pltpu.sync_copy(acc_sh.at[pl.ds(nbase, NPT)],
                        out_hbm.at[b, pl.ds(nbase, NPT)])
        plsc.subcore_barrier()

    @pl.when(cc == 0)
    def _():
        do_block(0)
        do_block(1)

    @pl.when(cc == 1)
    def _():
        do_block(2)
        do_block(3)


_agg4 = pl.kernel(
    _agg4_body,
    out_type=jax.ShapeDtypeStruct((4, NPAD, F_IN), jnp.float32),
    mesh=_SC_MESH, scratch_types=_AGG_SCRATCH)


def _pool_body(hh_hbm, batch_hbm, out_hbm, batch_v, rows_a, rows_b, acc_v,
               sem_a, sem_b):
    cc = lax.axis_index("c")
    ss = lax.axis_index("s")
    wid = ss * NC + cc
    g0 = wid * 2

    pltpu.sync_copy(batch_hbm, batch_v.at[pl.ds(0, N)])
    batch_v[pl.ds(N, 16)] = jnp.full((16,), G + 1, jnp.int32)

    def lower_bound(g):
        def bstep(i, lohi):
            lo, hi = lohi
            mid = (lo + hi) // 2
            v = batch_v[pl.ds(mid, 16)][0]
            lo2 = jnp.where(v < g, mid + 1, lo)
            hi2 = jnp.where(v < g, hi, mid)
            return lo2, hi2

        lo, _ = lax.fori_loop(0, 14, bstep, (0, N))
        return lo

    bounds = (lower_bound(g0), lower_bound(g0 + 1), lower_bound(g0 + 2))

    rows = (rows_a, rows_b)
    sems = (sem_a, sem_b)

    def win(j, w0):
        return jnp.minimum(w0 + j * CHP, N - CHP)

    def issue(j, p, w0):
        pltpu.async_copy(hh_hbm.at[pl.ds(win(j, w0), CHP)], rows[p], sems[p])

    def drain(p):
        pltpu.make_async_copy(hh_hbm.at[pl.ds(0, CHP)], rows[p], sems[p]).wait()

    def process(j, p, lo, hi, w0):
        eff = win(j, w0)
        r_lo = jnp.maximum(lo - eff, 0)
        r_hi = jnp.minimum(hi - eff, CHP)

        def row_step(r, c2):
            for q in range(QV):
                sl = pl.ds(q * 16, 16)
                acc_v[0, sl] = jnp.maximum(acc_v[0, sl], rows[p][r, sl])
            return c2

        lax.fori_loop(r_lo, r_hi, row_step, 0)

    for k in range(2):
        lo = bounds[k]
        hi = bounds[k + 1]
        for q in range(QV):
            acc_v[0, pl.ds(q * 16, 16)] = jnp.full((16,), -jnp.inf, jnp.float32)
        w0 = (lo // 8) * 8
        nch = (hi - w0 + CHP - 1) // CHP

        @pl.when(nch > 0)
        def _(lo=lo, hi=hi, w0=w0, nch=nch):
            issue(0, 0, w0)

            def pair_step(p2, carry):
                j0 = 2 * p2
                j1 = j0 + 1

                @pl.when(j1 < nch)
                def _():
                    issue(j1, 1, w0)

                drain(0)
                process(j0, 0, lo, hi, w0)

                @pl.when(j0 + 2 < nch)
                def _():
                    issue(j0 + 2, 0, w0)

                @pl.when(j1 < nch)
                def _():
                    drain(1)
                    process(j1, 1, lo, hi, w0)

                return carry

            lax.fori_loop(0, (nch + 1) // 2, pair_step, 0)

        pltpu.sync_copy(acc_v, out_hbm.at[g0 + k])


_pool = pl.kernel(
    _pool_body,
    out_type=jax.ShapeDtypeStruct((G, 1, 1024), jnp.float32),
    mesh=plsc.VectorSubcoreMesh(core_axis_name="c", subcore_axis_name="s",
                                num_cores=NC, num_subcores=NS),
    scratch_types=[
        pltpu.VMEM((N + 16,), jnp.int32),
        pltpu.VMEM((CHP, 1024), jnp.float32),
        pltpu.VMEM((CHP, 1024), jnp.float32),
        pltpu.VMEM((1, 1024), jnp.float32),
        pltpu.SemaphoreType.DMA,
        pltpu.SemaphoreType.DMA,
    ],
)


_R1 = 2000


def _bf(v):
    return v.astype(jnp.bfloat16)


def _tc1_body(parts_ref, x_ref, wrel_ref, wroot_ref, b_ref, out_ref):
    agg = parts_ref[0] + parts_ref[1]
    y = jnp.dot(_bf(agg), _bf(wrel_ref[...]),
                preferred_element_type=jnp.float32)
    y = y + jnp.dot(_bf(x_ref[...]), _bf(wroot_ref[...]),
                    preferred_element_type=jnp.float32)
    y = _leaky(y + b_ref[...])
    for j in range(4):
        out_ref[j] = y[:, j * 128:(j + 1) * 128]


def _tc1(parts1, x, W1_rel, W1_root, b1):
    return pl.pallas_call(
        _tc1_body,
        grid=(N // _R1,),
        in_specs=[
            pl.BlockSpec((2, _R1, 128), lambda i: (0, i, 0)),
            pl.BlockSpec((_R1, 128), lambda i: (i, 0)),
            pl.BlockSpec((F_IN, H), lambda i: (0, 0)),
            pl.BlockSpec((F_IN, H), lambda i: (0, 0)),
            pl.BlockSpec((1, H), lambda i: (0, 0)),
        ],
        out_specs=pl.BlockSpec((4, _R1, 128), lambda i: (0, i, 0)),
        out_shape=jax.ShapeDtypeStruct((4, N, 128), jnp.float32),
    )(parts1, x, W1_rel, W1_root, b1)


_R2 = 1000


def _tc2_body(x1b_ref, parts_ref, wrel_ref, wroot_ref, b2_ref,
              wl1_ref, bl1_ref, g_ref, be_ref, out_ref):
    acc = None
    for cb in range(4):
        aggc = parts_ref[cb]
        t = jnp.dot(_bf(aggc), _bf(wrel_ref[cb * 128:(cb + 1) * 128, :]),
                    preferred_element_type=jnp.float32)
        t = t + jnp.dot(_bf(x1b_ref[cb]),
                        _bf(wroot_ref[cb * 128:(cb + 1) * 128, :]),
                        preferred_element_type=jnp.float32)
        acc = t if acc is None else acc + t
    x2 = _leaky(acc + b2_ref[...])
    hacc = jnp.dot(_bf(x2), _bf(wl1_ref[512:1024, :]),
                   preferred_element_type=jnp.float32)
    for cb in range(4):
        hacc = hacc + jnp.dot(_bf(x1b_ref[cb]),
                              _bf(wl1_ref[cb * 128:(cb + 1) * 128, :]),
                              preferred_element_type=jnp.float32)
    hv = _leaky(hacc + bl1_ref[...])
    out_ref[...] = g_ref[...] * (hv * _INV_SQRT) + be_ref[...]


def _tc2(x1b, parts2, W2_rel, W2_root, b2, Wl1, bl1, g_l1, be_l1):
    return pl.pallas_call(
        _tc2_body,
        grid=(N // _R2,),
        in_specs=[
            pl.BlockSpec((4, _R2, 128), lambda i: (0, i, 0)),
            pl.BlockSpec((4, _R2, 128), lambda i: (0, i, 0)),
            pl.BlockSpec((H, H), lambda i: (0, 0)),
            pl.BlockSpec((H, H), lambda i: (0, 0)),
            pl.BlockSpec((1, H), lambda i: (0, 0)),
            pl.BlockSpec((2 * H, 1024), lambda i: (0, 0)),
            pl.BlockSpec((1, 1024), lambda i: (0, 0)),
            pl.BlockSpec((1, 1024), lambda i: (0, 0)),
            pl.BlockSpec((1, 1024), lambda i: (0, 0)),
        ],
        out_specs=pl.BlockSpec((_R2, 1024), lambda i: (i, 0)),
        out_shape=jax.ShapeDtypeStruct((N, 1024), jnp.float32),
    )(x1b, parts2, W2_rel, W2_root, b2, Wl1, bl1, g_l1, be_l1)


def _tc3_body(p_ref, wa_ref, ba_ref, ga_ref, bea_ref,
              wb_ref, bb_ref, gb_ref, beb_ref,
              wc_ref, bc_ref, gc_ref, bec_ref, out_ref):
    def bn(v, g, b):
        return g * (v * _INV_SQRT) + b

    o = bn(_leaky(jnp.dot(p_ref[...], wa_ref[...],
                          preferred_element_type=jnp.float32) + ba_ref[...]),
           ga_ref[...], bea_ref[...])
    o = bn(_leaky(jnp.dot(o, wb_ref[...],
                          preferred_element_type=jnp.float32) + bb_ref[...]),
           gb_ref[...], beb_ref[...])
    o = bn(_leaky(jnp.dot(o, wc_ref[...],
                          preferred_element_type=jnp.float32) + bc_ref[...]),
           gc_ref[...], bec_ref[...])
    out_ref[...] = o


def _tc3(pooled, Wa, ba, ga, bea, Wb, bb, gb, beb, Wcp, bcp, gcp, becp):
    return pl.pallas_call(
        _tc3_body,
        out_shape=jax.ShapeDtypeStruct((G, 128), jnp.float32),
    )(pooled, Wa, ba, ga, bea, Wb, bb, gb, beb, Wcp, bcp, gcp, becp)


def kernel(x, edge_index, batch, W1_rel, W1_root, b1, W2_rel, W2_root, b2,
           Wl1, bl1, g_l1, be_l1, Wa, ba, ga, bea, Wb, bb, gb, beb,
           Wc, bc, gc, bec):
    src2 = edge_index[0].reshape(NW, NCHUNK, CH)
    dst2 = edge_index[1].reshape(NW, NCHUNK, CH)
    zeros = jnp.zeros((NPT, F_IN), jnp.float32)

    parts1 = _agg1(x, src2, dst2, zeros)
    x1b = _tc1(parts1, x,
               W1_rel, W1_root, b1.reshape(1, H))
    parts2 = _agg4(x1b[0], x1b[1], x1b[2], x1b[3],
                   src2, dst2, zeros)
    hh = _tc2(x1b, parts2, W2_rel, W2_root, b2.reshape(1, H),
              Wl1, bl1.reshape(1, 1024), g_l1.reshape(1, 1024),
              be_l1.reshape(1, 1024))
    pooled = _pool(hh, batch).reshape(G, 1024)

    Wcp = jnp.pad(Wc, ((0, 0), (0, 128 - C)))
    bcp = jnp.pad(bc, (0, 128 - C)).reshape(1, 128)
    gcp = jnp.pad(gc, (0, 128 - C), constant_values=1.0).reshape(1, 128)
    becp = jnp.pad(bec, (0, 128 - C)).reshape(1, 128)
    o = _tc3(pooled, Wa, ba.reshape(1, 512), ga.reshape(1, 512),
             bea.reshape(1, 512), Wb, bb.reshape(1, 256), gb.reshape(1, 256),
             beb.reshape(1, 256), Wcp, bcp, gcp, becp)[:, :C]
    return (o, pooled)


# speedup vs baseline: 2.1068x; 1.0650x over previous
import functools
import math

import jax
import jax.numpy as jnp
from jax import lax
from jax.experimental import pallas as pl
from jax.experimental.pallas import tpu as pltpu
from jax.experimental.pallas import tpu_sc as plsc

N = 10000
E = 320000
F_IN = 128
H = 512
C = 10
G = 64

NC = 2
NS = 16
NW = NC * NS
EPW = E // NW
CH = 80
NCHUNK = EPW // CH
NPAD = 10240
NPT = NPAD // NS
CHP = 48
QV = 1024 // 16

_INV_SQRT = 1.0 / math.sqrt(1.0 + 1e-5)


def _leaky(v):
    return jnp.where(v >= 0, v, 0.01 * v)


_SC_MESH = plsc.VectorSubcoreMesh(core_axis_name="c", subcore_axis_name="s",
                                  num_cores=NC, num_subcores=NS)
_AGG_SCRATCH = [
    pltpu.VMEM((NCHUNK, CH), jnp.int32),
    pltpu.VMEM((NCHUNK, CH), jnp.int32),
    pltpu.VMEM((CH, F_IN), jnp.float32),
    pltpu.VMEM_SHARED((NPAD, F_IN), jnp.float32),
    pltpu.SemaphoreType.DMA,
]


def _edge_pass(tab, w, src_hbm, dst_hbm, acc_sh, src_v, dst_v, rows_v, sem):
    pltpu.sync_copy(src_hbm.at[w], src_v)
    pltpu.sync_copy(dst_hbm.at[w], dst_v)

    def step(i, carry):
        pltpu.async_copy(tab.at[src_v.at[i]], rows_v, sem).wait()
        pltpu.sync_copy(rows_v, acc_sh.at[dst_v.at[i]], add=True)
        return carry

    lax.fori_loop(0, NCHUNK, step, 0)


def _agg1_body(tab, src_hbm, dst_hbm, zeros_hbm, out_hbm,
               src_v, dst_v, rows_v, acc_sh, sem):
    cc = lax.axis_index("c")
    ss = lax.axis_index("s")
    wid = ss * NC + cc
    nbase = ss * NPT
    pltpu.sync_copy(zeros_hbm, acc_sh.at[pl.ds(nbase, NPT)])
    plsc.subcore_barrier()
    _edge_pass(tab, wid, src_hbm, dst_hbm, acc_sh, src_v, dst_v, rows_v, sem)
    plsc.subcore_barrier()
    pltpu.sync_copy(acc_sh.at[pl.ds(nbase, NPT)],
                    out_hbm.at[cc, pl.ds(nbase, NPT)])


_agg1 = pl.kernel(
    _agg1_body,
    out_type=jax.ShapeDtypeStruct((NC, NPAD, F_IN), jnp.float32),
    mesh=_SC_MESH, scratch_types=_AGG_SCRATCH)


def _agg4_body(t0, t1, t2, t3, src_hbm, dst_hbm, zeros_hbm, out_hbm,
               src_v, dst_v, rows_v, acc_sh, sem):
    cc = lax.axis_index("c")
    ss = lax.axis_index("s")
    nbase = ss * NPT
    tabs = (t0, t1, t2, t3)

    def do_block(b):
        tab = tabs[b]
        pltpu.sync_copy(zeros_hbm, acc_sh.at[pl.ds(nbase, NPT)])
        plsc.subcore_barrier()
        for half in range(2):
            _edge_pass(tab, ss * 2 + half, src_hbm, dst_hbm, acc_sh,
                       src_v, dst_v, rows_v, sem)
        plsc.subcore_barrier()
        pltpu.sync_copy(acc_sh.at[pl.ds(nbase, NPT)],
                        out_hbm.at[b, pl.ds(nbase, NPT)])
        plsc.subcore_barrier()

    @pl.when(cc == 0)
    def _():
        do_block(0)
        do_block(1)

    @pl.when(cc == 1)
    def _():
        do_block(2)
        do_block(3)


_agg4 = pl.kernel(
    _agg4_body,
    out_type=jax.ShapeDtypeStruct((4, NPAD, F_IN), jnp.float32),
    mesh=_SC_MESH, scratch_types=_AGG_SCRATCH)


def _pool_body(hh_hbm, batch_hbm, out_hbm, batch_v, rows_a, rows_b, acc_v,
               sem_a, sem_b):
    cc = lax.axis_index("c")
    ss = lax.axis_index("s")
    wid = ss * NC + cc
    g0 = wid * 2

    pltpu.sync_copy(batch_hbm, batch_v.at[pl.ds(0, N)])
    batch_v[pl.ds(N, 16)] = jnp.full((16,), G + 1, jnp.int32)

    def lower_bound(g):
        def bstep(i, lohi):
            lo, hi = lohi
            mid = (lo + hi) // 2
            v = batch_v[pl.ds(mid, 16)][0]
            lo2 = jnp.where(v < g, mid + 1, lo)
            hi2 = jnp.where(v < g, hi, mid)
            return lo2, hi2

        lo, _ = lax.fori_loop(0, 14, bstep, (0, N))
        return lo

    bounds = (lower_bound(g0), lower_bound(g0 + 1), lower_bound(g0 + 2))

    rows = (rows_a, rows_b)
    sems = (sem_a, sem_b)

    def win(j, w0):
        return jnp.minimum(w0 + j * CHP, N - CHP)

    def issue(j, p, w0):
        pltpu.async_copy(hh_hbm.at[pl.ds(win(j, w0), CHP)], rows[p], sems[p])

    def drain(p):
        pltpu.make_async_copy(hh_hbm.at[pl.ds(0, CHP)], rows[p], sems[p]).wait()

    def process(j, p, lo, hi, w0):
        eff = win(j, w0)
        r_lo = jnp.maximum(lo - eff, 0)
        r_hi = jnp.minimum(hi - eff, CHP)
        for qg in range(QV // 16):
            accs = tuple(acc_v[0, pl.ds((qg * 16 + q) * 16, 16)]
                         for q in range(16))

            def row_step(r, a):
                return tuple(
                    jnp.maximum(a[q], rows[p][r, pl.ds((qg * 16 + q) * 16, 16)])
                    for q in range(16))

            accs = lax.fori_loop(r_lo, r_hi, row_step, accs)
            for q in range(16):
                acc_v[0, pl.ds((qg * 16 + q) * 16, 16)] = accs[q]

    for k in range(2):
        lo = bounds[k]
        hi = bounds[k + 1]
        for q in range(QV):
            acc_v[0, pl.ds(q * 16, 16)] = jnp.full((16,), -jnp.inf, jnp.float32)
        w0 = (lo // 8) * 8
        nch = (hi - w0 + CHP - 1) // CHP

        @pl.when(nch > 0)
        def _(lo=lo, hi=hi, w0=w0, nch=nch):
            issue(0, 0, w0)

            def pair_step(p2, carry):
                j0 = 2 * p2
                j1 = j0 + 1

                @pl.when(j1 < nch)
                def _():
                    issue(j1, 1, w0)

                drain(0)
                process(j0, 0, lo, hi, w0)

                @pl.when(j0 + 2 < nch)
                def _():
                    issue(j0 + 2, 0, w0)

                @pl.when(j1 < nch)
                def _():
                    drain(1)
                    process(j1, 1, lo, hi, w0)

                return carry

            lax.fori_loop(0, (nch + 1) // 2, pair_step, 0)

        pltpu.sync_copy(acc_v, out_hbm.at[g0 + k])


_pool = pl.kernel(
    _pool_body,
    out_type=jax.ShapeDtypeStruct((G, 1, 1024), jnp.float32),
    mesh=plsc.VectorSubcoreMesh(core_axis_name="c", subcore_axis_name="s",
                                num_cores=NC, num_subcores=NS),
    scratch_types=[
        pltpu.VMEM((N + 16,), jnp.int32),
        pltpu.VMEM((CHP, 1024), jnp.float32),
        pltpu.VMEM((CHP, 1024), jnp.float32),
        pltpu.VMEM((1, 1024), jnp.float32),
        pltpu.SemaphoreType.DMA,
        pltpu.SemaphoreType.DMA,
    ],
)


_R1 = 2000


def _bf(v):
    return v.astype(jnp.bfloat16)


def _tc1_body(parts_ref, x_ref, wrel_ref, wroot_ref, b_ref, out_ref):
    agg = parts_ref[0] + parts_ref[1]
    y = jnp.dot(_bf(agg), _bf(wrel_ref[...]),
                preferred_element_type=jnp.float32)
    y = y + jnp.dot(_bf(x_ref[...]), _bf(wroot_ref[...]),
                    preferred_element_type=jnp.float32)
    y = _leaky(y + b_ref[...])
    for j in range(4):
        out_ref[j] = y[:, j * 128:(j + 1) * 128]


def _tc1(parts1, x, W1_rel, W1_root, b1):
    return pl.pallas_call(
        _tc1_body,
        grid=(N // _R1,),
        in_specs=[
            pl.BlockSpec((2, _R1, 128), lambda i: (0, i, 0)),
            pl.BlockSpec((_R1, 128), lambda i: (i, 0)),
            pl.BlockSpec((F_IN, H), lambda i: (0, 0)),
            pl.BlockSpec((F_IN, H), lambda i: (0, 0)),
            pl.BlockSpec((1, H), lambda i: (0, 0)),
        ],
        out_specs=pl.BlockSpec((4, _R1, 128), lambda i: (0, i, 0)),
        out_shape=jax.ShapeDtypeStruct((4, N, 128), jnp.float32),
    )(parts1, x, W1_rel, W1_root, b1)


_R2 = 1000


def _tc2_body(x1b_ref, parts_ref, wrel_ref, wroot_ref, b2_ref,
              wl1_ref, bl1_ref, g_ref, be_ref, out_ref):
    acc = None
    for cb in range(4):
        aggc = parts_ref[cb]
        t = jnp.dot(_bf(aggc), _bf(wrel_ref[cb * 128:(cb + 1) * 128, :]),
                    preferred_element_type=jnp.float32)
        t = t + jnp.dot(_bf(x1b_ref[cb]),
                        _bf(wroot_ref[cb * 128:(cb + 1) * 128, :]),
                        preferred_element_type=jnp.float32)
        acc = t if acc is None else acc + t
    x2 = _leaky(acc + b2_ref[...])
    hacc = jnp.dot(_bf(x2), _bf(wl1_ref[512:1024, :]),
                   preferred_element_type=jnp.float32)
    for cb in range(4):
        hacc = hacc + jnp.dot(_bf(x1b_ref[cb]),
                              _bf(wl1_ref[cb * 128:(cb + 1) * 128, :]),
                              preferred_element_type=jnp.float32)
    hv = _leaky(hacc + bl1_ref[...])
    out_ref[...] = g_ref[...] * (hv * _INV_SQRT) + be_ref[...]


def _tc2(x1b, parts2, W2_rel, W2_root, b2, Wl1, bl1, g_l1, be_l1):
    return pl.pallas_call(
        _tc2_body,
        grid=(N // _R2,),
        in_specs=[
            pl.BlockSpec((4, _R2, 128), lambda i: (0, i, 0)),
            pl.BlockSpec((4, _R2, 128), lambda i: (0, i, 0)),
            pl.BlockSpec((H, H), lambda i: (0, 0)),
            pl.BlockSpec((H, H), lambda i: (0, 0)),
            pl.BlockSpec((1, H), lambda i: (0, 0)),
            pl.BlockSpec((2 * H, 1024), lambda i: (0, 0)),
            pl.BlockSpec((1, 1024), lambda i: (0, 0)),
            pl.BlockSpec((1, 1024), lambda i: (0, 0)),
            pl.BlockSpec((1, 1024), lambda i: (0, 0)),
        ],
        out_specs=pl.BlockSpec((_R2, 1024), lambda i: (i, 0)),
        out_shape=jax.ShapeDtypeStruct((N, 1024), jnp.float32),
    )(x1b, parts2, W2_rel, W2_root, b2, Wl1, bl1, g_l1, be_l1)


def _tc3_body(p_ref, wa_ref, ba_ref, ga_ref, bea_ref,
              wb_ref, bb_ref, gb_ref, beb_ref,
              wc_ref, bc_ref, gc_ref, bec_ref, out_ref):
    def bn(v, g, b):
        return g * (v * _INV_SQRT) + b

    o = bn(_leaky(jnp.dot(p_ref[...], wa_ref[...],
                          preferred_element_type=jnp.float32) + ba_ref[...]),
           ga_ref[...], bea_ref[...])
    o = bn(_leaky(jnp.dot(o, wb_ref[...],
                          preferred_element_type=jnp.float32) + bb_ref[...]),
           gb_ref[...], beb_ref[...])
    o = bn(_leaky(jnp.dot(o, wc_ref[...],
                          preferred_element_type=jnp.float32) + bc_ref[...]),
           gc_ref[...], bec_ref[...])
    out_ref[...] = o


def _tc3(pooled, Wa, ba, ga, bea, Wb, bb, gb, beb, Wcp, bcp, gcp, becp):
    return pl.pallas_call(
        _tc3_body,
        out_shape=jax.ShapeDtypeStruct((G, 128), jnp.float32),
    )(pooled, Wa, ba, ga, bea, Wb, bb, gb, beb, Wcp, bcp, gcp, becp)


def kernel(x, edge_index, batch, W1_rel, W1_root, b1, W2_rel, W2_root, b2,
           Wl1, bl1, g_l1, be_l1, Wa, ba, ga, bea, Wb, bb, gb, beb,
           Wc, bc, gc, bec):
    src2 = edge_index[0].reshape(NW, NCHUNK, CH)
    dst2 = edge_index[1].reshape(NW, NCHUNK, CH)
    zeros = jnp.zeros((NPT, F_IN), jnp.float32)

    parts1 = _agg1(x, src2, dst2, zeros)
    x1b = _tc1(parts1, x,
               W1_rel, W1_root, b1.reshape(1, H))
    parts2 = _agg4(x1b[0], x1b[1], x1b[2], x1b[3],
                   src2, dst2, zeros)
    hh = _tc2(x1b, parts2, W2_rel, W2_root, b2.reshape(1, H),
              Wl1, bl1.reshape(1, 1024), g_l1.reshape(1, 1024),
              be_l1.reshape(1, 1024))
    pooled = _pool(hh, batch).reshape(G, 1024)

    Wcp = jnp.pad(Wc, ((0, 0), (0, 128 - C)))
    bcp = jnp.pad(bc, (0, 128 - C)).reshape(1, 128)
    gcp = jnp.pad(gc, (0, 128 - C), constant_values=1.0).reshape(1, 128)
    becp = jnp.pad(bec, (0, 128 - C)).reshape(1, 128)
    o = _tc3(pooled, Wa, ba.reshape(1, 512), ga.reshape(1, 512),
             bea.reshape(1, 512), Wb, bb.reshape(1, 256), gb.reshape(1, 256),
             beb.reshape(1, 256), Wcp, bcp, gcp, becp)[:, :C]
    return (o, pooled)
